# baseline (speedup 1.0000x reference)
import functools

import jax
import jax.numpy as jnp
from jax import lax
from jax.experimental import pallas as pl
from jax.experimental.pallas import tpu as pltpu
from jax.experimental.pallas import tpu_sc as plsc

N = 10000
E = 320000
NP = 10240
EP = 327680
NC = 2
NS = 16
NW = NC * NS
ET = EP // NW
IROWS = ET // 128
CB = 1024
CROWS = CB // 128
NCHUNK = ET // CB
NZ = NP // NS

_mesh = plsc.VectorSubcoreMesh(core_axis_name="c", subcore_axis_name="s")


@functools.partial(
    pl.kernel,
    out_type=jax.ShapeDtypeStruct((NC, NP), jnp.float32),
    mesh=_mesh,
    compiler_params=pltpu.CompilerParams(use_tc_tiling_on_sc=False),
    scratch_types=[
        pltpu.VMEM((IROWS, 128), jnp.int32),
        pltpu.VMEM((128,), jnp.float32),
        pltpu.VMEM_SHARED((NP,), jnp.float32),
        pltpu.SemaphoreType.DMA,
    ],
)
def _deg_kernel(dst_hbm, zeros_hbm, out_hbm, idx_v, ones_v, acc, sem):
    c = lax.axis_index("c")
    s = lax.axis_index("s")
    w = c * NS + s
    pltpu.sync_copy(dst_hbm.at[w], idx_v)
    for j in range(8):
        ones_v[pl.ds(j * 16, 16)] = jnp.ones((16,), jnp.float32)
    pltpu.sync_copy(zeros_hbm.at[pl.ds(s * NZ, NZ)], acc.at[pl.ds(s * NZ, NZ)])
    plsc.subcore_barrier()

    def chunk(g, carry):
        hs = []
        for j in range(CROWS):
            hs.append(pltpu.async_copy(
                ones_v, acc.at[idx_v.at[g * CROWS + j]], sem, add=True))
        for h in hs:
            h.wait()
        return carry

    lax.fori_loop(0, NCHUNK, chunk, 0)
    plsc.subcore_barrier()
    pltpu.sync_copy(acc.at[pl.ds(s * NZ, NZ)], out_hbm.at[c, pl.ds(s * NZ, NZ)])


def _make_agg(D):
    @functools.partial(
        pl.kernel,
        out_type=jax.ShapeDtypeStruct((NC, NP, D), jnp.float32),
        mesh=_mesh,
        compiler_params=pltpu.CompilerParams(use_tc_tiling_on_sc=False),
        scratch_types=[
            pltpu.VMEM((IROWS, 128), jnp.int32),
            pltpu.VMEM((IROWS, 128), jnp.int32),
            pltpu.VMEM((CB, D), jnp.float32),
            pltpu.VMEM_SHARED((NP, D), jnp.float32),
            pltpu.SemaphoreType.DMA,
            pltpu.SemaphoreType.DMA,
        ],
    )
    def _agg(hs_hbm, src_hbm, dst_hbm, zeros_hbm, out_hbm,
             isv, idv, rows, acc, gsem, ssem):
        c = lax.axis_index("c")
        s = lax.axis_index("s")
        w = c * NS + s
        pltpu.sync_copy(src_hbm.at[w], isv)
        pltpu.sync_copy(dst_hbm.at[w], idv)
        pltpu.sync_copy(zeros_hbm.at[pl.ds(s * NZ, NZ)],
                        acc.at[pl.ds(s * NZ, NZ)])
        plsc.subcore_barrier()

        def chunk(g, carry):
            hs = []
            for j in range(CROWS):
                hs.append(pltpu.async_copy(
                    hs_hbm.at[isv.at[g * CROWS + j]],
                    rows.at[pl.ds(j * 128, 128)], gsem))
            for h in hs:
                h.wait()
            sc = []
            for j in range(CROWS):
                sc.append(pltpu.async_copy(
                    rows.at[pl.ds(j * 128, 128)],
                    acc.at[idv.at[g * CROWS + j]], ssem, add=True))
            for h in sc:
                h.wait()
            return carry

        lax.fori_loop(0, NCHUNK, chunk, 0)
        plsc.subcore_barrier()
        pltpu.sync_copy(acc.at[pl.ds(s * NZ, NZ)],
                        out_hbm.at[c, pl.ds(s * NZ, NZ)])

    return _agg


_agg32 = _make_agg(32)
_agg16 = _make_agg(16)


ECB = 256
ECROWS = ECB // 128
ENCH = ET // ECB


@functools.partial(
    pl.kernel,
    out_type=jax.ShapeDtypeStruct((EP,), jnp.float32),
    mesh=_mesh,
    compiler_params=pltpu.CompilerParams(use_tc_tiling_on_sc=False,
                                         needs_layout_passes=False),
    scratch_types=[
        pltpu.VMEM((IROWS, 128), jnp.int32),
        pltpu.VMEM((IROWS, 128), jnp.int32),
        pltpu.VMEM((ECB, 64), jnp.float32),
        pltpu.VMEM((ECB, 64), jnp.float32),
        pltpu.VMEM((64, 16), jnp.float32),
        pltpu.VMEM((16,), jnp.float32),
        pltpu.VMEM((ET,), jnp.float32),
        pltpu.SemaphoreType.DMA,
    ],
)
def _edge_score(a_hbm, b_hbm, src_hbm, dst_hbm, m2_hbm, mb2_hbm, out_hbm,
                isv, idv, ra, rb, m2v, mb2v, outv, sem):
    c = lax.axis_index("c")
    s = lax.axis_index("s")
    w = c * NS + s
    pltpu.sync_copy(src_hbm.at[w], isv)
    pltpu.sync_copy(dst_hbm.at[w], idv)
    pltpu.sync_copy(m2_hbm, m2v)
    pltpu.sync_copy(mb2_hbm, mb2v)
    mb2loc = mb2v[...]

    def chunk(g, carry):
        hs = []
        for j in range(ECROWS):
            hs.append(pltpu.async_copy(
                a_hbm.at[isv.at[g * ECROWS + j]],
                ra.at[pl.ds(j * 128, 128)], sem))
            hs.append(pltpu.async_copy(
                b_hbm.at[idv.at[g * ECROWS + j]],
                rb.at[pl.ds(j * 128, 128)], sem))
        for h in hs:
            h.wait()

        def group(gg, carry2):
            rowv = gg * 16 + jnp.arange(16, dtype=jnp.int32)
            accs = [mb2loc] + [jnp.zeros((16,), jnp.float32)] * 7
            for step in range(8):
                for q in range(8):
                    k = q * 8 + step
                    colv = jnp.full((16,), k, jnp.int32)
                    av = plsc.load_gather(ra, [rowv, colv])
                    bv = plsc.load_gather(rb, [rowv, colv])
                    accs[q] = accs[q] + jnp.maximum(av + bv, 0.0) * m2v[k, :]
            r0 = (accs[0] + accs[1]) + (accs[2] + accs[3])
            r1 = (accs[4] + accs[5]) + (accs[6] + accs[7])
            outv[pl.ds(g * ECB + gg * 16, 16)] = r0 + r1
            return carry2

        lax.fori_loop(0, ECB // 16, group, 0)
        return carry

    lax.fori_loop(0, ENCH, chunk, 0)
    pltpu.sync_copy(outv, out_hbm.at[pl.ds(w * ET, ET)])


def _tc1_body(deg_ref, x_ref, w_ref, o_ref):
    dinv = lax.rsqrt(deg_ref[0, :] + deg_ref[1, :] + 1.0)
    h = jnp.dot(x_ref[...], w_ref[...], preferred_element_type=jnp.float32)
    o_ref[...] = h * dinv[:, None]


def _tc2_body(deg_ref, s1_ref, h1s_ref, w2_ref, b1_ref, o_ref):
    dinv = lax.rsqrt(deg_ref[0, :] + deg_ref[1, :] + 1.0)[:, None]
    pre = dinv * (s1_ref[0] + s1_ref[1] + h1s_ref[...]) + b1_ref[...]
    h1r = jnp.maximum(pre, 0.0)
    h2 = jnp.dot(h1r, w2_ref[...], preferred_element_type=jnp.float32)
    o_ref[...] = h2 * dinv


def _tc3_body(deg_ref, s2_ref, h2s_ref, b2_ref, m1s_ref, m1d_ref, mb1_ref,
              a_ref, b_ref):
    dinv = lax.rsqrt(deg_ref[0, :] + deg_ref[1, :] + 1.0)[:, None]
    h = dinv * (s2_ref[0] + s2_ref[1] + h2s_ref[...]) + b2_ref[...]
    a_ref[...] = jnp.dot(h, m1s_ref[...],
                         preferred_element_type=jnp.float32) + mb1_ref[...]
    b_ref[...] = jnp.dot(h, m1d_ref[...], preferred_element_type=jnp.float32)


def kernel(x, edge_index, W1, b1, W2, b2, M1, mb1, M2, mb2):
    src = edge_index[0].astype(jnp.int32)
    dst = edge_index[1].astype(jnp.int32)
    pad = jnp.full((EP - E,), N, jnp.int32)
    srcR = jnp.concatenate([src, pad]).reshape(NW, IROWS, 128)
    dstR = jnp.concatenate([dst, pad]).reshape(NW, IROWS, 128)
    xp = jnp.pad(x, ((0, NP - N), (0, 0)))
    W1p = jnp.pad(W1, ((0, 0), (0, 12)))
    b1p = jnp.pad(b1, (0, 12)).reshape(1, 32)
    W2p = jnp.pad(W2, ((0, 12), (0, 0)))
    b2r = b2.reshape(1, 16)
    z1 = jnp.zeros((NP,), jnp.float32)
    z32 = jnp.zeros((NP, 32), jnp.float32)
    z16 = jnp.zeros((NP, 16), jnp.float32)

    deg2 = _deg_kernel(dstR, z1)

    h1s = pl.pallas_call(
        _tc1_body,
        out_shape=jax.ShapeDtypeStruct((NP, 32), jnp.float32),
    )(deg2, xp, W1p)

    S1 = _agg32(h1s, srcR, dstR, z32)

    h2s = pl.pallas_call(
        _tc2_body,
        out_shape=jax.ShapeDtypeStruct((NP, 16), jnp.float32),
    )(deg2, S1, h1s, W2p, b1p)

    S2 = _agg16(h2s, srcR, dstR, z16)

    A, B = pl.pallas_call(
        _tc3_body,
        out_shape=(jax.ShapeDtypeStruct((NP, 64), jnp.float32),
                   jax.ShapeDtypeStruct((NP, 64), jnp.float32)),
    )(deg2, S2, h2s, b2r, M1[:16], M1[16:], mb1.reshape(1, 64))

    m2 = jnp.broadcast_to(M2, (64, 16))
    mb2b = jnp.broadcast_to(mb2, (16,))
    logits = _edge_score(A, B, srcR, dstR, m2, mb2b)

    return logits[:E].reshape(E, 1)


# speedup vs baseline: 2.7577x; 2.7577x over previous
import functools

import jax
import jax.numpy as jnp
from jax import lax
from jax.experimental import pallas as pl
from jax.experimental.pallas import tpu as pltpu
from jax.experimental.pallas import tpu_sc as plsc

N = 10000
E = 320000
NP = 10240
EP = 327680
NC = 2
NS = 16
NW = NC * NS
ET = EP // NW
IROWS = ET // 128
CB = 1024
CROWS = CB // 128
NCHUNK = ET // CB
NZ = NP // NS
GROWS = EP * 16 // 128
CGR = CB * 16 // 128

_mesh = plsc.VectorSubcoreMesh(core_axis_name="c", subcore_axis_name="s")


@functools.partial(
    pl.kernel,
    out_type=jax.ShapeDtypeStruct((NC, NP), jnp.float32),
    mesh=_mesh,
    compiler_params=pltpu.CompilerParams(use_tc_tiling_on_sc=False),
    scratch_types=[
        pltpu.VMEM((IROWS, 128), jnp.int32),
        pltpu.VMEM((128,), jnp.float32),
        pltpu.VMEM_SHARED((NP,), jnp.float32),
        pltpu.SemaphoreType.DMA,
    ],
)
def _deg_kernel(dst_hbm, zeros_hbm, out_hbm, idx_v, ones_v, acc, sem):
    c = lax.axis_index("c")
    s = lax.axis_index("s")
    w = c * NS + s
    pltpu.sync_copy(dst_hbm.at[w], idx_v)
    for j in range(8):
        ones_v[pl.ds(j * 16, 16)] = jnp.ones((16,), jnp.float32)
    pltpu.sync_copy(zeros_hbm.at[pl.ds(s * NZ, NZ)], acc.at[pl.ds(s * NZ, NZ)])
    plsc.subcore_barrier()

    def chunk(g, carry):
        hs = []
        for j in range(CROWS):
            hs.append(pltpu.async_copy(
                ones_v, acc.at[idx_v.at[g * CROWS + j]], sem, add=True))
        for h in hs:
            h.wait()
        return carry

    lax.fori_loop(0, NCHUNK, chunk, 0)
    plsc.subcore_barrier()
    pltpu.sync_copy(acc.at[pl.ds(s * NZ, NZ)], out_hbm.at[c, pl.ds(s * NZ, NZ)])


def _make_agg(D):
    @functools.partial(
        pl.kernel,
        out_type=jax.ShapeDtypeStruct((NC, NP, D), jnp.float32),
        mesh=_mesh,
        compiler_params=pltpu.CompilerParams(use_tc_tiling_on_sc=False),
        scratch_types=[
            pltpu.VMEM((IROWS, 128), jnp.int32),
            pltpu.VMEM((IROWS, 128), jnp.int32),
            pltpu.VMEM((2, CB, D), jnp.float32),
            pltpu.VMEM_SHARED((NP, D), jnp.float32),
            pltpu.SemaphoreType.DMA,
            pltpu.SemaphoreType.DMA,
        ],
    )
    def _agg(hs_hbm, src_hbm, dst_hbm, zeros_hbm, out_hbm,
             isv, idv, rows, acc, gsem, ssem):
        c = lax.axis_index("c")
        s = lax.axis_index("s")
        w = c * NS + s
        pltpu.sync_copy(src_hbm.at[w], isv)
        pltpu.sync_copy(dst_hbm.at[w], idv)
        pltpu.sync_copy(zeros_hbm.at[pl.ds(s * NZ, NZ)],
                        acc.at[pl.ds(s * NZ, NZ)])
        plsc.subcore_barrier()

        def fire_g(g, b):
            for j in range(CROWS):
                pltpu.async_copy(
                    hs_hbm.at[isv.at[g * CROWS + j]],
                    rows.at[b, pl.ds(j * 128, 128)], gsem)

        def drain_g(g, b):
            for j in range(CROWS):
                pltpu.make_async_copy(
                    hs_hbm.at[isv.at[g * CROWS + j]],
                    rows.at[b, pl.ds(j * 128, 128)], gsem).wait()

        def fire_s(g, b):
            for j in range(CROWS):
                pltpu.async_copy(
                    rows.at[b, pl.ds(j * 128, 128)],
                    acc.at[idv.at[g * CROWS + j]], ssem, add=True)

        def drain_s(g, b):
            for j in range(CROWS):
                pltpu.make_async_copy(
                    rows.at[b, pl.ds(j * 128, 128)],
                    acc.at[idv.at[g * CROWS + j]], ssem).wait()

        fire_g(0, 0)

        def chunk(g, carry):
            b = lax.rem(g, 2)

            @pl.when(g + 1 < NCHUNK)
            def _fire_next():
                fire_g(g + 1, 1 - b)

            drain_g(g, b)
            fire_s(g, b)
            drain_s(g, b)
            return carry

        lax.fori_loop(0, NCHUNK, chunk, 0)
        plsc.subcore_barrier()
        pltpu.sync_copy(acc.at[pl.ds(s * NZ, NZ)],
                        out_hbm.at[c, pl.ds(s * NZ, NZ)])

    return _agg


_agg32 = _make_agg(32)
_agg16 = _make_agg(16)


@functools.partial(
    pl.kernel,
    out_type=(jax.ShapeDtypeStruct((GROWS, 128), jnp.float32),
              jax.ShapeDtypeStruct((GROWS, 128), jnp.float32)),
    mesh=_mesh,
    compiler_params=pltpu.CompilerParams(use_tc_tiling_on_sc=False),
    scratch_types=[
        pltpu.VMEM((IROWS, 128), jnp.int32),
        pltpu.VMEM((IROWS, 128), jnp.int32),
        pltpu.VMEM((CB, 16), jnp.float32),
        pltpu.VMEM((CGR, 128), jnp.float32),
        pltpu.SemaphoreType.DMA,
    ],
)
def _edge_gather(h_hbm, src_hbm, dst_hbm, gs_hbm, gd_hbm,
                 isv, idv, rows, packed, sem):
    c = lax.axis_index("c")
    s = lax.axis_index("s")
    w = c * NS + s
    pltpu.sync_copy(src_hbm.at[w], isv)
    pltpu.sync_copy(dst_hbm.at[w], idv)
    base = w * (ET * 16 // 128)

    def repack(r, carry):
        for c0 in range(8):
            packed[r, pl.ds(c0 * 16, 16)] = rows[r * 8 + c0, :]
        return carry

    def chunk(g, carry):
        off = base + g * CGR
        hs = []
        for j in range(CROWS):
            hs.append(pltpu.async_copy(
                h_hbm.at[isv.at[g * CROWS + j]],
                rows.at[pl.ds(j * 128, 128)], sem))
        for h in hs:
            h.wait()
        lax.fori_loop(0, CGR, repack, 0)
        pltpu.sync_copy(packed, gs_hbm.at[pl.ds(off, CGR)])
        hd = []
        for j in range(CROWS):
            hd.append(pltpu.async_copy(
                h_hbm.at[idv.at[g * CROWS + j]],
                rows.at[pl.ds(j * 128, 128)], sem))
        for h in hd:
            h.wait()
        lax.fori_loop(0, CGR, repack, 0)
        pltpu.sync_copy(packed, gd_hbm.at[pl.ds(off, CGR)])
        return carry

    lax.fori_loop(0, NCHUNK, chunk, 0)


def _tc1_body(deg_ref, x_ref, w_ref, o_ref):
    dinv = lax.rsqrt(deg_ref[0, :] + deg_ref[1, :] + 1.0)
    h = jnp.dot(x_ref[...], w_ref[...], preferred_element_type=jnp.float32)
    o_ref[...] = h * dinv[:, None]


def _tc2_body(deg_ref, s1_ref, h1s_ref, w2_ref, b1_ref, o_ref):
    dinv = lax.rsqrt(deg_ref[0, :] + deg_ref[1, :] + 1.0)[:, None]
    pre = dinv * (s1_ref[0] + s1_ref[1] + h1s_ref[...]) + b1_ref[...]
    h1r = jnp.maximum(pre, 0.0)
    h2 = jnp.dot(h1r, w2_ref[...], preferred_element_type=jnp.float32)
    o_ref[...] = h2 * dinv


def _tc3_body(deg_ref, s2_ref, h2s_ref, b2_ref, o_ref):
    dinv = lax.rsqrt(deg_ref[0, :] + deg_ref[1, :] + 1.0)[:, None]
    o_ref[...] = dinv * (s2_ref[0] + s2_ref[1] + h2s_ref[...]) + b2_ref[...]


BR = 2048


def _mlp_body(mb2_ref, gs_ref, gd_ref, ms_ref, md_ref, mb1_ref, m2_ref, k_ref,
              o_ref):
    hid = jnp.dot(gs_ref[...], ms_ref[...], preferred_element_type=jnp.float32)
    hid = hid + jnp.dot(gd_ref[...], md_ref[...],
                        preferred_element_type=jnp.float32)
    hid = jnp.maximum(hid + mb1_ref[...], 0.0)
    t = hid * m2_ref[...]
    l8 = jnp.dot(t, k_ref[...], preferred_element_type=jnp.float32)
    o_ref[...] = l8 + mb2_ref[0]


def kernel(x, edge_index, W1, b1, W2, b2, M1, mb1, M2, mb2):
    src = edge_index[0].astype(jnp.int32)
    dst = edge_index[1].astype(jnp.int32)
    pad = jnp.full((EP - E,), N, jnp.int32)
    srcR = jnp.concatenate([src, pad]).reshape(NW, IROWS, 128)
    dstR = jnp.concatenate([dst, pad]).reshape(NW, IROWS, 128)
    xp = jnp.pad(x, ((0, NP - N), (0, 0)))
    W1p = jnp.pad(W1, ((0, 0), (0, 12)))
    b1p = jnp.pad(b1, (0, 12)).reshape(1, 32)
    W2p = jnp.pad(W2, ((0, 12), (0, 0)))
    b2r = b2.reshape(1, 16)
    z1 = jnp.zeros((NP,), jnp.float32)
    z32 = jnp.zeros((NP, 32), jnp.float32)
    z16 = jnp.zeros((NP, 16), jnp.float32)

    deg2 = _deg_kernel(dstR, z1)

    h1s = pl.pallas_call(
        _tc1_body,
        out_shape=jax.ShapeDtypeStruct((NP, 32), jnp.float32),
    )(deg2, xp, W1p)

    S1 = _agg32(h1s, srcR, dstR, z32)

    h2s = pl.pallas_call(
        _tc2_body,
        out_shape=jax.ShapeDtypeStruct((NP, 16), jnp.float32),
    )(deg2, S1, h1s, W2p, b1p)

    S2 = _agg16(h2s, srcR, dstR, z16)

    h = pl.pallas_call(
        _tc3_body,
        out_shape=jax.ShapeDtypeStruct((NP, 16), jnp.float32),
    )(deg2, S2, h2s, b2r)

    gs, gd = _edge_gather(h, srcR, dstR)

    eye8 = jnp.eye(8, dtype=jnp.float32)
    ms = jnp.kron(eye8, M1[:16])
    md = jnp.kron(eye8, M1[16:])
    mb1t = jnp.tile(mb1, 8).reshape(1, 512)
    m2t = jnp.tile(M2[:, 0], 8).reshape(1, 512)
    ksum = jnp.kron(eye8, jnp.ones((64, 1), jnp.float32))

    logits = pl.pallas_call(
        _mlp_body,
        grid=(GROWS // BR,),
        in_specs=[
            pl.BlockSpec(memory_space=pltpu.SMEM),
            pl.BlockSpec((BR, 128), lambda i: (i, 0)),
            pl.BlockSpec((BR, 128), lambda i: (i, 0)),
            pl.BlockSpec((128, 512), lambda i: (0, 0)),
            pl.BlockSpec((128, 512), lambda i: (0, 0)),
            pl.BlockSpec((1, 512), lambda i: (0, 0)),
            pl.BlockSpec((1, 512), lambda i: (0, 0)),
            pl.BlockSpec((512, 8), lambda i: (0, 0)),
        ],
        out_specs=pl.BlockSpec((BR, 8), lambda i: (i, 0)),
        out_shape=jax.ShapeDtypeStruct((GROWS, 8), jnp.float32),
    )(mb2, gs, gd, ms, md, mb1t, m2t, ksum)

    return logits.reshape(EP)[:E].reshape(E, 1)


# speedup vs baseline: 2.9219x; 1.0595x over previous
import functools

import jax
import jax.numpy as jnp
from jax import lax
from jax.experimental import pallas as pl
from jax.experimental.pallas import tpu as pltpu
from jax.experimental.pallas import tpu_sc as plsc

N = 10000
E = 320000
NP = 10240
EP = 327680
NC = 2
NS = 16
NW = NC * NS
ET = EP // NW
IROWS = ET // 128
CB = 1024
CROWS = CB // 128
NCHUNK = ET // CB
NZ = NP // NS
GROWS = EP * 16 // 128
CGR = CB * 16 // 128

_mesh = plsc.VectorSubcoreMesh(core_axis_name="c", subcore_axis_name="s")


@functools.partial(
    pl.kernel,
    out_type=jax.ShapeDtypeStruct((NC, NP), jnp.float32),
    mesh=_mesh,
    compiler_params=pltpu.CompilerParams(use_tc_tiling_on_sc=False),
    scratch_types=[
        pltpu.VMEM((IROWS, 128), jnp.int32),
        pltpu.VMEM((128,), jnp.float32),
        pltpu.VMEM_SHARED((NP,), jnp.float32),
        pltpu.SemaphoreType.DMA,
    ],
)
def _deg_kernel(dst_hbm, zeros_hbm, out_hbm, idx_v, ones_v, acc, sem):
    c = lax.axis_index("c")
    s = lax.axis_index("s")
    w = c * NS + s
    pltpu.sync_copy(dst_hbm.at[w], idx_v)
    for j in range(8):
        ones_v[pl.ds(j * 16, 16)] = jnp.ones((16,), jnp.float32)
    pltpu.sync_copy(zeros_hbm.at[pl.ds(s * NZ, NZ)], acc.at[pl.ds(s * NZ, NZ)])
    plsc.subcore_barrier()

    def chunk(g, carry):
        hs = []
        for j in range(CROWS):
            hs.append(pltpu.async_copy(
                ones_v, acc.at[idx_v.at[g * CROWS + j]], sem, add=True))
        for h in hs:
            h.wait()
        return carry

    lax.fori_loop(0, NCHUNK, chunk, 0)
    plsc.subcore_barrier()
    pltpu.sync_copy(acc.at[pl.ds(s * NZ, NZ)], out_hbm.at[c, pl.ds(s * NZ, NZ)])


def _make_agg(D, CBA, NB):
    NCH = ET // CBA
    CRW = CBA // 128

    @functools.partial(
        pl.kernel,
        out_type=jax.ShapeDtypeStruct((NC, NP, D), jnp.float32),
        mesh=_mesh,
        compiler_params=pltpu.CompilerParams(use_tc_tiling_on_sc=False),
        scratch_types=[
            pltpu.VMEM((IROWS, 128), jnp.int32),
            pltpu.VMEM((IROWS, 128), jnp.int32),
            pltpu.VMEM((NB, CBA, D), jnp.float32),
            pltpu.VMEM_SHARED((NP, D), jnp.float32),
            pltpu.SemaphoreType.DMA,
            pltpu.SemaphoreType.DMA,
        ],
    )
    def _agg(hs_hbm, src_hbm, dst_hbm, zeros_hbm, out_hbm,
             isv, idv, rows, acc, gsem, ssem):
        c = lax.axis_index("c")
        s = lax.axis_index("s")
        w = c * NS + s
        pltpu.sync_copy(src_hbm.at[w], isv)
        pltpu.sync_copy(dst_hbm.at[w], idv)
        pltpu.sync_copy(zeros_hbm.at[pl.ds(s * NZ, NZ)],
                        acc.at[pl.ds(s * NZ, NZ)])
        plsc.subcore_barrier()

        def fire_g(g, b):
            for j in range(CRW):
                pltpu.async_copy(
                    hs_hbm.at[isv.at[g * CRW + j]],
                    rows.at[b, pl.ds(j * 128, 128)], gsem)

        def drain_g(g, b):
            for j in range(CRW):
                pltpu.make_async_copy(
                    hs_hbm.at[isv.at[g * CRW + j]],
                    rows.at[b, pl.ds(j * 128, 128)], gsem).wait()

        def fire_s(g, b):
            for j in range(CRW):
                pltpu.async_copy(
                    rows.at[b, pl.ds(j * 128, 128)],
                    acc.at[idv.at[g * CRW + j]], ssem, add=True)

        def drain_s(g, b):
            for j in range(CRW):
                pltpu.make_async_copy(
                    rows.at[b, pl.ds(j * 128, 128)],
                    acc.at[idv.at[g * CRW + j]], ssem).wait()

        for p in range(NB - 1):
            fire_g(p, p)

        def chunk(g, carry):
            b = lax.rem(g, NB)

            @pl.when(g + NB - 1 < NCH)
            def _fire_next():
                fire_g(g + NB - 1, lax.rem(g + NB - 1, NB))

            drain_g(g, b)
            fire_s(g, b)
            drain_s(g, b)
            return carry

        lax.fori_loop(0, NCH, chunk, 0)
        plsc.subcore_barrier()
        pltpu.sync_copy(acc.at[pl.ds(s * NZ, NZ)],
                        out_hbm.at[c, pl.ds(s * NZ, NZ)])

    return _agg


_agg32 = _make_agg(32, 512, 4)
_agg16 = _make_agg(16, 1024, 3)


@functools.partial(
    pl.kernel,
    out_type=(jax.ShapeDtypeStruct((GROWS, 128), jnp.float32),
              jax.ShapeDtypeStruct((GROWS, 128), jnp.float32)),
    mesh=_mesh,
    compiler_params=pltpu.CompilerParams(use_tc_tiling_on_sc=False),
    scratch_types=[
        pltpu.VMEM((IROWS, 128), jnp.int32),
        pltpu.VMEM((IROWS, 128), jnp.int32),
        pltpu.VMEM((2, CB, 16), jnp.float32),
        pltpu.VMEM((2, CGR, 128), jnp.float32),
        pltpu.SemaphoreType.DMA,
        pltpu.SemaphoreType.DMA,
        pltpu.SemaphoreType.DMA,
        pltpu.SemaphoreType.DMA,
    ],
)
def _edge_gather(h_hbm, src_hbm, dst_hbm, gs_hbm, gd_hbm,
                 isv, idv, rows, packed, sem0, sem1, ws0, ws1):
    c = lax.axis_index("c")
    s = lax.axis_index("s")
    w = c * NS + s
    pltpu.sync_copy(src_hbm.at[w], isv)
    pltpu.sync_copy(dst_hbm.at[w], idv)
    base = w * (ET * 16 // 128)

    def fire(idx, g, b, sem):
        for j in range(CROWS):
            pltpu.async_copy(
                h_hbm.at[idx.at[g * CROWS + j]],
                rows.at[b, pl.ds(j * 128, 128)], sem)

    def drain(idx, g, b, sem):
        for j in range(CROWS):
            pltpu.make_async_copy(
                h_hbm.at[idx.at[g * CROWS + j]],
                rows.at[b, pl.ds(j * 128, 128)], sem).wait()

    def repack(b):
        def body(r, carry):
            for c0 in range(8):
                packed[b, r, pl.ds(c0 * 16, 16)] = rows[b, r * 8 + c0, :]
            return carry
        lax.fori_loop(0, CGR, body, 0)

    def fire_w(out_hbm, g, b, sem):
        pltpu.async_copy(packed.at[b], out_hbm.at[pl.ds(base + g * CGR, CGR)],
                         sem)

    def drain_w(out_hbm, g, b, sem):
        pltpu.make_async_copy(packed.at[b],
                              out_hbm.at[pl.ds(base + g * CGR, CGR)],
                              sem).wait()

    fire(isv, 0, 0, sem0)

    def chunk(g, carry):
        fire(idv, g, 1, sem1)
        drain(isv, g, 0, sem0)

        @pl.when(g > 0)
        def _w0():
            drain_w(gs_hbm, g - 1, 0, ws0)

        repack(0)
        fire_w(gs_hbm, g, 0, ws0)

        @pl.when(g + 1 < NCHUNK)
        def _g0():
            fire(isv, g + 1, 0, sem0)

        drain(idv, g, 1, sem1)

        @pl.when(g > 0)
        def _w1():
            drain_w(gd_hbm, g - 1, 1, ws1)

        repack(1)
        fire_w(gd_hbm, g, 1, ws1)
        return carry

    lax.fori_loop(0, NCHUNK, chunk, 0)
    drain_w(gs_hbm, NCHUNK - 1, 0, ws0)
    drain_w(gd_hbm, NCHUNK - 1, 1, ws1)


def _tc1_body(deg_ref, x_ref, w_ref, o_ref):
    dinv = lax.rsqrt(deg_ref[0, :] + deg_ref[1, :] + 1.0)
    h = jnp.dot(x_ref[...], w_ref[...], preferred_element_type=jnp.float32)
    o_ref[...] = h * dinv[:, None]


def _tc2_body(deg_ref, s1_ref, h1s_ref, w2_ref, b1_ref, o_ref):
    dinv = lax.rsqrt(deg_ref[0, :] + deg_ref[1, :] + 1.0)[:, None]
    pre = dinv * (s1_ref[0] + s1_ref[1] + h1s_ref[...]) + b1_ref[...]
    h1r = jnp.maximum(pre, 0.0)
    h2 = jnp.dot(h1r, w2_ref[...], preferred_element_type=jnp.float32)
    o_ref[...] = h2 * dinv


def _tc3_body(deg_ref, s2_ref, h2s_ref, b2_ref, o_ref):
    dinv = lax.rsqrt(deg_ref[0, :] + deg_ref[1, :] + 1.0)[:, None]
    o_ref[...] = dinv * (s2_ref[0] + s2_ref[1] + h2s_ref[...]) + b2_ref[...]


BR = 2048


def _mlp_body(mb2_ref, gs_ref, gd_ref, ms_ref, md_ref, mb1_ref, m2_ref, k_ref,
              o_ref):
    hid = jnp.dot(gs_ref[...], ms_ref[...], preferred_element_type=jnp.float32)
    hid = hid + jnp.dot(gd_ref[...], md_ref[...],
                        preferred_element_type=jnp.float32)
    hid = jnp.maximum(hid + mb1_ref[...], 0.0)
    t = hid * m2_ref[...]
    l8 = jnp.dot(t, k_ref[...], preferred_element_type=jnp.float32)
    o_ref[...] = l8 + mb2_ref[0]


def kernel(x, edge_index, W1, b1, W2, b2, M1, mb1, M2, mb2):
    src = edge_index[0].astype(jnp.int32)
    dst = edge_index[1].astype(jnp.int32)
    pad = jnp.full((EP - E,), N, jnp.int32)
    srcR = jnp.concatenate([src, pad]).reshape(NW, IROWS, 128)
    dstR = jnp.concatenate([dst, pad]).reshape(NW, IROWS, 128)
    xp = jnp.pad(x, ((0, NP - N), (0, 0)))
    W1p = jnp.pad(W1, ((0, 0), (0, 12)))
    b1p = jnp.pad(b1, (0, 12)).reshape(1, 32)
    W2p = jnp.pad(W2, ((0, 12), (0, 0)))
    b2r = b2.reshape(1, 16)
    z1 = jnp.zeros((NP,), jnp.float32)
    z32 = jnp.zeros((NP, 32), jnp.float32)
    z16 = jnp.zeros((NP, 16), jnp.float32)

    deg2 = _deg_kernel(dstR, z1)

    h1s = pl.pallas_call(
        _tc1_body,
        out_shape=jax.ShapeDtypeStruct((NP, 32), jnp.float32),
    )(deg2, xp, W1p)

    S1 = _agg32(h1s, srcR, dstR, z32)

    h2s = pl.pallas_call(
        _tc2_body,
        out_shape=jax.ShapeDtypeStruct((NP, 16), jnp.float32),
    )(deg2, S1, h1s, W2p, b1p)

    S2 = _agg16(h2s, srcR, dstR, z16)

    h = pl.pallas_call(
        _tc3_body,
        out_shape=jax.ShapeDtypeStruct((NP, 16), jnp.float32),
    )(deg2, S2, h2s, b2r)

    gs, gd = _edge_gather(h, srcR, dstR)

    eye8 = jnp.eye(8, dtype=jnp.float32)
    ms = jnp.kron(eye8, M1[:16])
    md = jnp.kron(eye8, M1[16:])
    mb1t = jnp.tile(mb1, 8).reshape(1, 512)
    m2t = jnp.tile(M2[:, 0], 8).reshape(1, 512)
    ksum = jnp.kron(eye8, jnp.ones((64, 1), jnp.float32))

    logits = pl.pallas_call(
        _mlp_body,
        grid=(GROWS // BR,),
        in_specs=[
            pl.BlockSpec(memory_space=pltpu.SMEM),
            pl.BlockSpec((BR, 128), lambda i: (i, 0)),
            pl.BlockSpec((BR, 128), lambda i: (i, 0)),
            pl.BlockSpec((128, 512), lambda i: (0, 0)),
            pl.BlockSpec((128, 512), lambda i: (0, 0)),
            pl.BlockSpec((1, 512), lambda i: (0, 0)),
            pl.BlockSpec((1, 512), lambda i: (0, 0)),
            pl.BlockSpec((512, 8), lambda i: (0, 0)),
        ],
        out_specs=pl.BlockSpec((BR, 8), lambda i: (i, 0)),
        out_shape=jax.ShapeDtypeStruct((GROWS, 8), jnp.float32),
    )(mb2, gs, gd, ms, md, mb1t, m2t, ksum)

    return logits.reshape(EP)[:E].reshape(E, 1)
